# baseline (speedup 1.0000x reference)
import functools

import jax
import jax.numpy as jnp
from jax import lax
from jax.experimental import pallas as pl
from jax.experimental.pallas import tpu as pltpu
from jax.experimental.pallas import tpu_sc as plsc

_NC = 2
_NS = 16
_NW = _NC * _NS

_B = 4096
_L = 200
_D = 64
_V = 1000000
_BLK = _B // _NW
_NBUF = 4
_NTF = _V // 128
_VP = _V // 2
_CSTEPS = 246


def _repack_body(tokt_hbm, tail_hbm, out_hbm, sbuf, wbuf,
                 isem0, isem1, osem0, osem1):
    wid = lax.axis_index("s") * _NC + lax.axis_index("c")
    isems = (isem0, isem1)
    osems = (osem0, osem1)
    lanes = lax.iota(jnp.int32, 16)
    rvecs = [8 * j + (lanes >> 1) for j in range(8)]
    c64 = (lanes & 1) * 64

    def fire_in(ci, slot):
        t = wid + 32 * ci
        @pl.when(t < _NTF)
        def _full():
            pltpu.async_copy(tokt_hbm.at[:, pl.ds(t * 128, 128)],
                             sbuf.at[slot], isems[slot])

    def substep(ci, u):
        t = wid + 32 * ci
        @pl.when(t == _NTF)
        def _tail():
            pltpu.sync_copy(tail_hbm, sbuf.at[u, pl.ds(0, 32)])
            pltpu.sync_copy(sbuf.at[u, pl.ds(0, 32)],
                            out_hbm.at[pl.ds(_NTF * _D, 32)])
        @pl.when(t < _NTF)
        def _work():
            fire_in(ci + 1, u ^ 1)
            pltpu.make_async_copy(tokt_hbm.at[:, pl.ds(0, 128)],
                                  sbuf.at[u], isems[u]).wait()
            @pl.when(ci >= 2)
            def _reclaim():
                pltpu.make_async_copy(wbuf.at[u, :, pl.ds(0, 128)],
                                      out_hbm.at[pl.ds(0, _D)],
                                      osems[u]).wait()
            @plsc.parallel_loop(0, _D, 1, unroll=2)
            def _rows(d):
                cvec = c64 + jnp.full((16,), d, dtype=jnp.int32)
                for j in range(8):
                    v = sbuf[u, d, pl.ds(16 * j, 16)]
                    plsc.store_scatter(wbuf.at[u], [rvecs[j], cvec], v)

            pltpu.async_copy(wbuf.at[u, :, pl.ds(0, 128)],
                             out_hbm.at[pl.ds(t * _D, _D)], osems[u])

    fire_in(0, 0)

    def step(i, _):
        substep(2 * i, 0)
        substep(2 * i + 1, 1)
        return _

    lax.fori_loop(0, _CSTEPS // 2, step, 0, unroll=False)
    for u in range(2):
        pltpu.make_async_copy(wbuf.at[u, :, pl.ds(0, 128)],
                              out_hbm.at[pl.ds(0, _D)], osems[u]).wait()


@jax.jit
def _repack(tokt, tail):
    kfn = functools.partial(
        pl.kernel,
        out_type=jax.ShapeDtypeStruct((_VP, 128), jnp.float32),
        mesh=plsc.VectorSubcoreMesh(core_axis_name="c", subcore_axis_name="s"),
        scratch_types=[
            pltpu.VMEM((2, _D, 128), jnp.float32),
            pltpu.VMEM((2, _D, 130), jnp.float32),
            pltpu.SemaphoreType.DMA,
            pltpu.SemaphoreType.DMA,
            pltpu.SemaphoreType.DMA,
            pltpu.SemaphoreType.DMA,
        ],
        compiler_params=pltpu.CompilerParams(needs_layout_passes=False),
    )(_repack_body)
    return kfn(tokt, tail)


def _sc_body(xt_hbm, tok2_hbm, pos_hbm, out_hbm,
             idx_v, idxg, pos_v, gbuf0, gbuf1, gbuf2, gbuf3, tbuf,
             gsem0, gsem1, gsem2, gsem3, osem0, osem1):
    wid = lax.axis_index("s") * _NC + lax.axis_index("c")
    b0 = wid * _BLK

    pltpu.sync_copy(xt_hbm.at[:, pl.ds(b0, _BLK)], idx_v)
    pltpu.sync_copy(pos_hbm, pos_v)

    gbufs = (gbuf0, gbuf1, gbuf2, gbuf3)
    gsems = (gsem0, gsem1, gsem2, gsem3)
    osems = (osem0, osem1)

    lanes = lax.iota(jnp.int32, 16)
    dvecs = [16 * k + lanes for k in range(4)]

    def fire_gather(l, slot):
        @pl.when(l < _L)
        def _():
            for j in range(8):
                sl = pl.ds(16 * j, 16)
                idxg[slot, sl] = idx_v[l, sl] >> 1
            pltpu.async_copy(tok2_hbm.at[idxg.at[slot]], gbufs[slot],
                             gsems[slot])

    def substep(l, u):
        fire_gather(l + _NBUF - 1, (u + _NBUF - 1) % _NBUF)
        pv = [pos_v[l, pl.ds(16 * k, 16)] for k in range(4)]
        @pl.when(l >= 2)
        def _reclaim():
            for s in range(8):
                pltpu.make_async_copy(
                    tbuf.at[u % 2, pl.ds(8 * s, 8), pl.ds(0, _BLK)],
                    out_hbm.at[0, s, wid], osems[u % 2]).wait()
        pltpu.make_async_copy(tok2_hbm.at[idxg.at[u]], gbufs[u],
                              gsems[u]).wait()
        dst = tbuf.at[u % 2]
        gb = gbufs[u]

        @plsc.parallel_loop(0, _BLK // 16, 1, unroll=1)
        def _rows(jj):
            poffs = (idx_v[l, pl.ds(16 * jj, 16)] & 1) << 6
            for m in range(16):
                b = 16 * jj + m
                poff = poffs[m]
                bvec = jnp.full((16,), b, dtype=jnp.int32)
                for k in range(4):
                    v = gb[b, pl.ds(poff + 16 * k, 16)] + pv[k]
                    plsc.store_scatter(dst, [dvecs[k], bvec], v)

        for s in range(8):
            pltpu.async_copy(
                tbuf.at[u % 2, pl.ds(8 * s, 8), pl.ds(0, _BLK)],
                out_hbm.at[l, s, wid], osems[u % 2])

    for l in range(_NBUF - 1):
        fire_gather(l, l)

    def step(i, _):
        for u in range(_NBUF):
            substep(_NBUF * i + u, u)
        return _

    lax.fori_loop(0, _L // _NBUF, step, 0, unroll=False)
    for u in range(2):
        for s in range(8):
            pltpu.make_async_copy(
                tbuf.at[u, pl.ds(8 * s, 8), pl.ds(0, _BLK)],
                out_hbm.at[0, s, wid], osems[u]).wait()


@jax.jit
def _tok_pos_embed(xt, tok2, pos_table):
    kfn = functools.partial(
        pl.kernel,
        out_type=jax.ShapeDtypeStruct((_L, 8, _NW, 8, _BLK), jnp.float32),
        mesh=plsc.VectorSubcoreMesh(core_axis_name="c", subcore_axis_name="s"),
        scratch_types=[
            pltpu.VMEM((_L, _BLK), jnp.int32),
            pltpu.VMEM((_NBUF, _BLK), jnp.int32),
            pltpu.VMEM((_L, _D), jnp.float32),
            pltpu.VMEM((_BLK, 2 * _D), jnp.float32),
            pltpu.VMEM((_BLK, 2 * _D), jnp.float32),
            pltpu.VMEM((_BLK, 2 * _D), jnp.float32),
            pltpu.VMEM((_BLK, 2 * _D), jnp.float32),
            pltpu.VMEM((2, _D, 129), jnp.float32),
            pltpu.SemaphoreType.DMA,
            pltpu.SemaphoreType.DMA,
            pltpu.SemaphoreType.DMA,
            pltpu.SemaphoreType.DMA,
            pltpu.SemaphoreType.DMA,
            pltpu.SemaphoreType.DMA,
        ],
        compiler_params=pltpu.CompilerParams(use_tc_tiling_on_sc=False,
                                             needs_layout_passes=False),
    )(_sc_body)
    return kfn(xt, tok2, pos_table)


def kernel(x, token_table, pos_table):
    xt = x.astype(jnp.int32).T
    tail = token_table[_V - _D:].reshape(32, 128)
    tok2 = _repack(token_table.T, tail)
    out5 = _tok_pos_embed(xt, tok2, pos_table)
    return out5.transpose(2, 4, 0, 1, 3).reshape(_B, _L, _D)


# speedup vs baseline: 1.7633x; 1.7633x over previous
import functools

import numpy as np

import jax
import jax.numpy as jnp
from jax import lax
from jax.experimental import pallas as pl
from jax.experimental.pallas import tpu as pltpu
from jax.experimental.pallas import tpu_sc as plsc

_NC = 2
_NS = 16
_NW = _NC * _NS

_B = 4096
_L = 200
_D = 64
_BLK = _B // _NW
_NBUF = 4


def _sc_body(xt_hbm, tok2_hbm, pos_hbm, out_hbm,
             idx_v, pos_v, gbuf0, gbuf1, gbuf2, gbuf3, tbuf,
             gsem0, gsem1, gsem2, gsem3, osem0, osem1):
    wid = lax.axis_index("s") * _NC + lax.axis_index("c")
    b0 = wid * _BLK
    tok_hbm = tok2_hbm

    pltpu.sync_copy(xt_hbm.at[:, pl.ds(b0, _BLK)], idx_v)
    pltpu.sync_copy(pos_hbm, pos_v)

    gbufs = (gbuf0, gbuf1, gbuf2, gbuf3)
    gsems = (gsem0, gsem1, gsem2, gsem3)
    osems = (osem0, osem1)

    lanes = lax.iota(jnp.int32, 16)
    dvecs = [16 * k + lanes for k in range(4)]

    def fire_gather(l, slot):
        @pl.when(l < _L)
        def _():
            pltpu.async_copy(tok_hbm.at[idx_v.at[l]], gbufs[slot],
                             gsems[slot])

    def substep(l, u):
        fire_gather(l + _NBUF - 1, (u + _NBUF - 1) % _NBUF)
        pv = [pos_v[l, pl.ds(16 * k, 16)] for k in range(4)]
        @pl.when(l >= 2)
        def _reclaim():
            for s in range(8):
                pltpu.make_async_copy(
                    tbuf.at[u % 2, pl.ds(8 * s, 8), pl.ds(0, _BLK)],
                    out_hbm.at[0, s, wid], osems[u % 2]).wait()
        pltpu.make_async_copy(tok_hbm.at[idx_v.at[l]], gbufs[u],
                              gsems[u]).wait()
        dst = tbuf.at[u % 2]
        gb = gbufs[u]

        @plsc.parallel_loop(0, _BLK, 1, unroll=8)
        def _rows(b):
            bvec = jnp.full((16,), b, dtype=jnp.int32)
            for k in range(4):
                v = gb[b, pl.ds(16 * k, 16)] + pv[k]
                plsc.store_scatter(dst, [dvecs[k], bvec], v)

        for s in range(8):
            pltpu.async_copy(
                tbuf.at[u % 2, pl.ds(8 * s, 8), pl.ds(0, _BLK)],
                out_hbm.at[l, s, wid], osems[u % 2])

    for l in range(_NBUF - 1):
        fire_gather(l, l)

    def step(i, _):
        for u in range(_NBUF):
            substep(_NBUF * i + u, u)
        return _

    lax.fori_loop(0, _L // _NBUF, step, 0, unroll=False)
    for u in range(2):
        for s in range(8):
            pltpu.make_async_copy(
                tbuf.at[u, pl.ds(8 * s, 8), pl.ds(0, _BLK)],
                out_hbm.at[0, s, wid], osems[u]).wait()


@jax.jit
def _tok_pos_embed(xt, token_table, pos_table):
    kfn = functools.partial(
        pl.kernel,
        out_type=jax.ShapeDtypeStruct((_L, 8, _NW, 8, _BLK), jnp.float32),
        mesh=plsc.VectorSubcoreMesh(core_axis_name="c", subcore_axis_name="s"),
        scratch_types=[
            pltpu.VMEM((_L, _BLK), jnp.int32),
            pltpu.VMEM((_L, _D), jnp.float32),
            pltpu.VMEM((_BLK, _D), jnp.float32),
            pltpu.VMEM((_BLK, _D), jnp.float32),
            pltpu.VMEM((_BLK, _D), jnp.float32),
            pltpu.VMEM((_BLK, _D), jnp.float32),
            pltpu.VMEM((2, _D, 129), jnp.float32),
            pltpu.SemaphoreType.DMA,
            pltpu.SemaphoreType.DMA,
            pltpu.SemaphoreType.DMA,
            pltpu.SemaphoreType.DMA,
            pltpu.SemaphoreType.DMA,
            pltpu.SemaphoreType.DMA,
        ],
        compiler_params=pltpu.CompilerParams(use_tc_tiling_on_sc=False,
                                             needs_layout_passes=False),
    )(_sc_body)
    return kfn(xt, token_table, pos_table)


def kernel(x, token_table, pos_table):
    xt = x.astype(jnp.int32).T
    out5 = _tok_pos_embed(xt, token_table, pos_table)
    return out5.transpose(2, 4, 0, 1, 3).reshape(_B, _L, _D)
